# baseline (speedup 1.0000x reference)
import functools

import jax
import jax.numpy as jnp
from jax import lax
from jax.experimental import pallas as pl
from jax.experimental.pallas import tpu as pltpu
from jax.experimental.pallas import tpu_sc as plsc

N = 100000
E = 3200000
WEIGHT = 0.01
EPS = 1e-08

NC = 2
NSUB = 16
NW = NC * NSUB
L = 16

NP = 100352
NODES_W = NP // NW
NODES_S = NP // NSUB
TROWS_S = N // NSUB
EP = NW * 100352
EW = EP // NW
CHUNKS = 98
SUBQ = 8
SW = 16
NS_ITERS = 20
MAGIC = 1597463007

_mesh = plsc.VectorSubcoreMesh(core_axis_name="c", subcore_axis_name="s",
                               num_cores=NC, num_subcores=NSUB)
_cparams = pltpu.CompilerParams(needs_layout_passes=False,
                                use_tc_tiling_on_sc=False)


def _rsqrt(x, iters=3):
    y = plsc.bitcast(MAGIC - (plsc.bitcast(x, jnp.int32) >> 1), jnp.float32)
    for _ in range(iters):
        u = x * y
        y = y * (1.5 - 0.5 * (u * y))
    return y


def _edge_body(t_hbm, i2d, j2d, z2d, s_part, acc_part,
               idx_i, idx_j, rows_i0, rows_i1, rows_j0, rows_j1,
               upd0, upd1, accbuf, s_sh,
               sem_g0, sem_g1, sem_s0, sem_s1):
    c = lax.axis_index("c")
    s = lax.axis_index("s")
    wid = c * NSUB + s
    iota = lax.iota(jnp.int32, L)
    zv = jnp.zeros((L,), jnp.float32)
    cols = [jnp.full((L,), k, jnp.int32) for k in (0, 1, 2, 3, 4, 5)]
    ocols = [jnp.full((L,), k, jnp.int32) for k in range(9)]
    rows_i = (rows_i0, rows_i1)
    rows_j = (rows_j0, rows_j1)
    upd = (upd0, upd1)
    sem_g = (sem_g0, sem_g1)
    sem_s = (sem_s0, sem_s1)

    def zloop(t, carry):
        pltpu.sync_copy(z2d, s_sh.at[pl.ds(s * NODES_S + t * 128, 128)])
        return carry
    lax.fori_loop(0, NODES_S // 128, zloop, 0)
    plsc.subcore_barrier()

    ebase = wid * EW
    rowbase = wid * (EW // 128)

    def _issue_gather(q):
        p = q & 1
        return (pltpu.async_copy(t_hbm.at[idx_i.at[q]], rows_i[p], sem_g[p]),
                pltpu.async_copy(t_hbm.at[idx_j.at[q]], rows_j[p], sem_g[p]))

    def chunk_body(cc, accs):
        pltpu.sync_copy(i2d.at[pl.ds(rowbase + cc * SUBQ, SUBQ)], idx_i)
        pltpu.sync_copy(j2d.at[pl.ds(rowbase + cc * SUBQ, SUBQ)], idx_j)
        gd = {0: _issue_gather(0), 1: _issue_gather(1)}
        sd = {}
        for q in range(SUBQ):
            p = q & 1
            gi, gj = gd.pop(q)
            gi.wait()
            gj.wait()
            if q >= 2:
                sd.pop(q - 2).wait()
            ri, rj, up = rows_i[p], rows_j[p], upd[p]
            esub = ebase + cc * (SUBQ * 128) + q * 128

            @plsc.parallel_loop(0, 8, unroll=4, carry=accs)
            def group(g, accs2):
                a1, a2 = accs2
                row = g * L + iota
                gvi = [plsc.load_gather(ri, [row, cl]) for cl in cols]
                gvj = [plsc.load_gather(rj, [row, cl]) for cl in cols]
                r0 = gvj[0] - gvi[0]
                r1 = gvj[1] - gvi[1]
                r2 = gvj[2] - gvi[2]
                d0 = gvj[3] - gvi[3]
                d1 = gvj[4] - gvi[4]
                d2 = gvj[5] - gvi[5]
                n2 = r0 * r0 + r1 * r1 + r2 * r2
                m2 = d0 * d0 + d1 * d1 + d2 * d2
                ln = n2 * _rsqrt(n2, 2)
                w = 1.0 / (ln + EPS)
                w = jnp.where((esub + g * L + iota) < E, w, 0.0)
                a1 = a1 + w * (n2 + m2)
                a2 = a2 + w
                wd0 = w * d0
                wd1 = w * d1
                wd2 = w * d2
                outs = (wd0 * r0, wd0 * r1, wd0 * r2,
                        wd1 * r0, wd1 * r1, wd1 * r2,
                        wd2 * r0, wd2 * r1, wd2 * r2)
                for k in range(9):
                    plsc.store_scatter(up, [row, ocols[k]], outs[k])
                return (a1, a2)

            accs = group
            if q < SUBQ - 2:
                gd[q + 2] = _issue_gather(q + 2)
            sd[q] = pltpu.async_copy(up, s_sh.at[idx_i.at[q]], sem_s[p],
                                     add=True)
        sd.pop(SUBQ - 2).wait()
        sd.pop(SUBQ - 1).wait()
        return accs

    acc1, acc2 = lax.fori_loop(0, CHUNKS, chunk_body, (zv, zv))

    plsc.subcore_barrier()
    pltpu.sync_copy(s_sh.at[pl.ds(s * NODES_S, NODES_S)],
                    s_part.at[c, pl.ds(s * NODES_S, NODES_S)])
    accbuf[0, :] = acc1
    accbuf[1, :] = acc2
    pltpu.sync_copy(accbuf, acc_part.at[c, s])


_edge_kernel = functools.partial(
    pl.kernel,
    out_type=(jax.ShapeDtypeStruct((NC, NP, SW), jnp.float32),
              jax.ShapeDtypeStruct((NC, NSUB, 2, L), jnp.float32)),
    mesh=_mesh,
    scratch_types=[
        pltpu.VMEM((SUBQ, 128), jnp.int32),
        pltpu.VMEM((SUBQ, 128), jnp.int32),
        pltpu.VMEM((128, 16), jnp.float32),
        pltpu.VMEM((128, 16), jnp.float32),
        pltpu.VMEM((128, 16), jnp.float32),
        pltpu.VMEM((128, 16), jnp.float32),
        pltpu.VMEM((128, SW), jnp.float32),
        pltpu.VMEM((128, SW), jnp.float32),
        pltpu.VMEM((2, L), jnp.float32),
        pltpu.VMEM_SHARED((NP, SW), jnp.float32),
        pltpu.SemaphoreType.DMA,
        pltpu.SemaphoreType.DMA,
        pltpu.SemaphoreType.DMA,
        pltpu.SemaphoreType.DMA,
    ],
    compiler_params=_cparams,
)(_edge_body)


def _polar_body(s_part, dot_part, buf0, buf1, ovec):
    c = lax.axis_index("c")
    s = lax.axis_index("s")
    wid = c * NSUB + s
    nbase = wid * NODES_W
    iota = lax.iota(jnp.int32, L)
    ocols = [jnp.full((L,), k, jnp.int32) for k in range(9)]
    dot0 = jnp.zeros((L,), jnp.float32)

    def chunk(cc, dot0):
        pltpu.sync_copy(s_part.at[0, pl.ds(nbase + cc * 784, 784)], buf0)
        pltpu.sync_copy(s_part.at[1, pl.ds(nbase + cc * 784, 784)], buf1)

        @plsc.parallel_loop(0, 49, unroll=2, carry=dot0)
        def group(g, dot):
            row = g * L + iota
            sv = [plsc.load_gather(buf0, [row, cl]) +
                  plsc.load_gather(buf1, [row, cl]) for cl in ocols]
            m = jnp.abs(sv[0])
            for k in range(1, 9):
                m = jnp.maximum(m, jnp.abs(sv[k]))
            scale = 1.0 / (4.0 * m + 1e-35)
            x = tuple(v * scale for v in sv)

            def ns(t, x):
                x0, x1, x2, x3, x4, x5, x6, x7, x8 = x
                z00 = x0 * x0 + x3 * x3 + x6 * x6
                z01 = x0 * x1 + x3 * x4 + x6 * x7
                z02 = x0 * x2 + x3 * x5 + x6 * x8
                z11 = x1 * x1 + x4 * x4 + x7 * x7
                z12 = x1 * x2 + x4 * x5 + x7 * x8
                z22 = x2 * x2 + x5 * x5 + x8 * x8
                t00 = 1.5 - 0.5 * z00
                t01 = -0.5 * z01
                t02 = -0.5 * z02
                t11 = 1.5 - 0.5 * z11
                t12 = -0.5 * z12
                t22 = 1.5 - 0.5 * z22
                return (x0 * t00 + x1 * t01 + x2 * t02,
                        x0 * t01 + x1 * t11 + x2 * t12,
                        x0 * t02 + x1 * t12 + x2 * t22,
                        x3 * t00 + x4 * t01 + x5 * t02,
                        x3 * t01 + x4 * t11 + x5 * t12,
                        x3 * t02 + x4 * t12 + x5 * t22,
                        x6 * t00 + x7 * t01 + x8 * t02,
                        x6 * t01 + x7 * t11 + x8 * t12,
                        x6 * t02 + x7 * t12 + x8 * t22)

            x0, x1, x2, x3, x4, x5, x6, x7, x8 = lax.fori_loop(
                0, NS_ITERS, ns, x)
            det = (x0 * (x4 * x8 - x5 * x7)
                   - x1 * (x3 * x8 - x5 * x6)
                   + x2 * (x3 * x7 - x4 * x6))
            neg = det < 0.0
            x0 = jnp.where(neg, -x0, x0)
            x3 = jnp.where(neg, -x3, x3)
            x6 = jnp.where(neg, -x6, x6)
            contrib = (x0 * sv[0] + x1 * sv[1] + x2 * sv[2]
                       + x3 * sv[3] + x4 * sv[4] + x5 * sv[5]
                       + x6 * sv[6] + x7 * sv[7] + x8 * sv[8])
            return dot + contrib

        return group

    dot = lax.fori_loop(0, NODES_W // 784, chunk, dot0)
    ovec[...] = dot
    pltpu.sync_copy(ovec, dot_part.at[c, s])


_polar_kernel = functools.partial(
    pl.kernel,
    out_type=jax.ShapeDtypeStruct((NC, NSUB, L), jnp.float32),
    mesh=_mesh,
    scratch_types=[
        pltpu.VMEM((784, SW), jnp.float32),
        pltpu.VMEM((784, SW), jnp.float32),
        pltpu.VMEM((L,), jnp.float32),
    ],
    compiler_params=_cparams,
)(_polar_body)


def kernel(mu0, mu, edge_idx):
    t = jnp.concatenate(
        [mu0, mu, jnp.zeros((N, 10), jnp.float32)], axis=1)
    pad = EP - E
    i2d = jnp.pad(edge_idx[0], (0, pad)).reshape(EP // 128, 128)
    j2d = jnp.pad(edge_idx[1], (0, pad)).reshape(EP // 128, 128)
    z2d = jnp.zeros((128, SW), jnp.float32)
    s_part, acc_part = _edge_kernel(t, i2d, j2d, z2d)
    dot_part = _polar_kernel(s_part)
    acc1 = jnp.sum(acc_part[:, :, 0, :])
    acc2 = jnp.sum(acc_part[:, :, 1, :])
    dot = jnp.sum(dot_part)
    return (WEIGHT * (acc1 - 2.0 * dot) / acc2).astype(jnp.float32)


# speedup vs baseline: 1.0737x; 1.0737x over previous
import functools

import jax
import jax.numpy as jnp
from jax import lax
from jax.experimental import pallas as pl
from jax.experimental.pallas import tpu as pltpu
from jax.experimental.pallas import tpu_sc as plsc

N = 100000
E = 3200000
WEIGHT = 0.01
EPS = 1e-08

NC = 2
NSUB = 16
NW = NC * NSUB
L = 16

NP = 100352
NODES_W = NP // NW
NODES_S = NP // NSUB
TROWS_S = N // NSUB
EP = NW * 100352
EW = EP // NW
CHUNKS = 98
SUBQ = 8
SW = 16
NS_ITERS = 20
MAGIC = 1597463007

_mesh = plsc.VectorSubcoreMesh(core_axis_name="c", subcore_axis_name="s",
                               num_cores=NC, num_subcores=NSUB)
_cparams = pltpu.CompilerParams(needs_layout_passes=False,
                                use_tc_tiling_on_sc=False)


def _rsqrt(x, iters=3):
    y = plsc.bitcast(MAGIC - (plsc.bitcast(x, jnp.int32) >> 1), jnp.float32)
    for _ in range(iters):
        u = x * y
        y = y * (1.5 - 0.5 * (u * y))
    return y


def _edge_body(t_hbm, i2d, j2d, z2d, s_part, acc_part,
               idx_i, idx_j, rows_i0, rows_i1, rows_j0, rows_j1,
               upd0, upd1, accbuf, s_sh,
               sem_g0, sem_g1, sem_s0, sem_s1):
    c = lax.axis_index("c")
    s = lax.axis_index("s")
    wid = c * NSUB + s
    iota = lax.iota(jnp.int32, L)
    zv = jnp.zeros((L,), jnp.float32)
    cols = [jnp.full((L,), k, jnp.int32) for k in (0, 1, 2, 3, 4, 5)]
    ocols = [jnp.full((L,), k, jnp.int32) for k in range(9)]
    rows_i = (rows_i0, rows_i1)
    rows_j = (rows_j0, rows_j1)
    upd = (upd0, upd1)
    sem_g = (sem_g0, sem_g1)
    sem_s = (sem_s0, sem_s1)

    def zloop(t, carry):
        pltpu.sync_copy(z2d, s_sh.at[pl.ds(s * NODES_S + t * 128, 128)])
        return carry
    lax.fori_loop(0, NODES_S // 128, zloop, 0)
    plsc.subcore_barrier()

    ebase = wid * EW
    rowbase = wid * (EW // 128)

    def _issue_gather(q):
        p = q & 1
        return (pltpu.async_copy(t_hbm.at[idx_i.at[q]], rows_i[p], sem_g[p]),
                pltpu.async_copy(t_hbm.at[idx_j.at[q]], rows_j[p], sem_g[p]))

    def chunk_body(cc, accs):
        pltpu.sync_copy(i2d.at[pl.ds(rowbase + cc * SUBQ, SUBQ)], idx_i)
        pltpu.sync_copy(j2d.at[pl.ds(rowbase + cc * SUBQ, SUBQ)], idx_j)
        gd = {0: _issue_gather(0), 1: _issue_gather(1)}
        sd = {}
        for q in range(SUBQ):
            p = q & 1
            gi, gj = gd.pop(q)
            gi.wait()
            gj.wait()
            if q >= 2:
                sd.pop(q - 2).wait()
            ri, rj, up = rows_i[p], rows_j[p], upd[p]
            esub = ebase + cc * (SUBQ * 128) + q * 128

            @plsc.parallel_loop(0, 8, unroll=4, carry=accs)
            def group(g, accs2):
                a1, a2 = accs2
                row = g * L + iota
                gvi = [plsc.load_gather(ri, [row, cl]) for cl in cols]
                gvj = [plsc.load_gather(rj, [row, cl]) for cl in cols]
                r0 = gvj[0] - gvi[0]
                r1 = gvj[1] - gvi[1]
                r2 = gvj[2] - gvi[2]
                d0 = gvj[3] - gvi[3]
                d1 = gvj[4] - gvi[4]
                d2 = gvj[5] - gvi[5]
                n2 = r0 * r0 + r1 * r1 + r2 * r2
                m2 = d0 * d0 + d1 * d1 + d2 * d2
                ln = n2 * _rsqrt(n2, 2)
                w = 1.0 / (ln + EPS)
                w = jnp.where((esub + g * L + iota) < E, w, 0.0)
                a1 = a1 + w * (n2 + m2)
                a2 = a2 + w
                wd0 = w * d0
                wd1 = w * d1
                wd2 = w * d2
                outs = (wd0 * r0, wd0 * r1, wd0 * r2,
                        wd1 * r0, wd1 * r1, wd1 * r2,
                        wd2 * r0, wd2 * r1, wd2 * r2)
                for k in range(9):
                    plsc.store_scatter(up, [row, ocols[k]], outs[k])
                return (a1, a2)

            accs = group
            if q < SUBQ - 2:
                gd[q + 2] = _issue_gather(q + 2)
            sd[q] = pltpu.async_copy(up, s_sh.at[idx_i.at[q]], sem_s[p],
                                     add=True)
        sd.pop(SUBQ - 2).wait()
        sd.pop(SUBQ - 1).wait()
        return accs

    acc1, acc2 = lax.fori_loop(0, CHUNKS, chunk_body, (zv, zv))

    plsc.subcore_barrier()
    pltpu.sync_copy(s_sh.at[pl.ds(s * NODES_S, NODES_S)],
                    s_part.at[c, pl.ds(s * NODES_S, NODES_S)])
    accbuf[0, :] = acc1
    accbuf[1, :] = acc2
    pltpu.sync_copy(accbuf, acc_part.at[c, s])


_edge_kernel = functools.partial(
    pl.kernel,
    out_type=(jax.ShapeDtypeStruct((NC, NP, SW), jnp.float32),
              jax.ShapeDtypeStruct((NC, NSUB, 2, L), jnp.float32)),
    mesh=_mesh,
    scratch_types=[
        pltpu.VMEM((SUBQ, 128), jnp.int32),
        pltpu.VMEM((SUBQ, 128), jnp.int32),
        pltpu.VMEM((128, 8), jnp.float32),
        pltpu.VMEM((128, 8), jnp.float32),
        pltpu.VMEM((128, 8), jnp.float32),
        pltpu.VMEM((128, 8), jnp.float32),
        pltpu.VMEM((128, SW), jnp.float32),
        pltpu.VMEM((128, SW), jnp.float32),
        pltpu.VMEM((2, L), jnp.float32),
        pltpu.VMEM_SHARED((NP, SW), jnp.float32),
        pltpu.SemaphoreType.DMA,
        pltpu.SemaphoreType.DMA,
        pltpu.SemaphoreType.DMA,
        pltpu.SemaphoreType.DMA,
    ],
    compiler_params=_cparams,
)(_edge_body)


def _polar_body(s_part, dot_part, buf0, buf1, ovec):
    c = lax.axis_index("c")
    s = lax.axis_index("s")
    wid = c * NSUB + s
    nbase = wid * NODES_W
    iota = lax.iota(jnp.int32, L)
    ocols = [jnp.full((L,), k, jnp.int32) for k in range(9)]
    dot0 = jnp.zeros((L,), jnp.float32)

    def chunk(cc, dot0):
        pltpu.sync_copy(s_part.at[0, pl.ds(nbase + cc * 784, 784)], buf0)
        pltpu.sync_copy(s_part.at[1, pl.ds(nbase + cc * 784, 784)], buf1)

        @plsc.parallel_loop(0, 49, unroll=2, carry=dot0)
        def group(g, dot):
            row = g * L + iota
            sv = [plsc.load_gather(buf0, [row, cl]) +
                  plsc.load_gather(buf1, [row, cl]) for cl in ocols]
            m = jnp.abs(sv[0])
            for k in range(1, 9):
                m = jnp.maximum(m, jnp.abs(sv[k]))
            scale = 1.0 / (4.0 * m + 1e-35)
            x = tuple(v * scale for v in sv)

            def ns(t, x):
                x0, x1, x2, x3, x4, x5, x6, x7, x8 = x
                z00 = x0 * x0 + x3 * x3 + x6 * x6
                z01 = x0 * x1 + x3 * x4 + x6 * x7
                z02 = x0 * x2 + x3 * x5 + x6 * x8
                z11 = x1 * x1 + x4 * x4 + x7 * x7
                z12 = x1 * x2 + x4 * x5 + x7 * x8
                z22 = x2 * x2 + x5 * x5 + x8 * x8
                t00 = 1.5 - 0.5 * z00
                t01 = -0.5 * z01
                t02 = -0.5 * z02
                t11 = 1.5 - 0.5 * z11
                t12 = -0.5 * z12
                t22 = 1.5 - 0.5 * z22
                return (x0 * t00 + x1 * t01 + x2 * t02,
                        x0 * t01 + x1 * t11 + x2 * t12,
                        x0 * t02 + x1 * t12 + x2 * t22,
                        x3 * t00 + x4 * t01 + x5 * t02,
                        x3 * t01 + x4 * t11 + x5 * t12,
                        x3 * t02 + x4 * t12 + x5 * t22,
                        x6 * t00 + x7 * t01 + x8 * t02,
                        x6 * t01 + x7 * t11 + x8 * t12,
                        x6 * t02 + x7 * t12 + x8 * t22)

            x0, x1, x2, x3, x4, x5, x6, x7, x8 = lax.fori_loop(
                0, NS_ITERS, ns, x)
            det = (x0 * (x4 * x8 - x5 * x7)
                   - x1 * (x3 * x8 - x5 * x6)
                   + x2 * (x3 * x7 - x4 * x6))
            neg = det < 0.0
            x0 = jnp.where(neg, -x0, x0)
            x3 = jnp.where(neg, -x3, x3)
            x6 = jnp.where(neg, -x6, x6)
            contrib = (x0 * sv[0] + x1 * sv[1] + x2 * sv[2]
                       + x3 * sv[3] + x4 * sv[4] + x5 * sv[5]
                       + x6 * sv[6] + x7 * sv[7] + x8 * sv[8])
            return dot + contrib

        return group

    dot = lax.fori_loop(0, NODES_W // 784, chunk, dot0)
    ovec[...] = dot
    pltpu.sync_copy(ovec, dot_part.at[c, s])


_polar_kernel = functools.partial(
    pl.kernel,
    out_type=jax.ShapeDtypeStruct((NC, NSUB, L), jnp.float32),
    mesh=_mesh,
    scratch_types=[
        pltpu.VMEM((784, SW), jnp.float32),
        pltpu.VMEM((784, SW), jnp.float32),
        pltpu.VMEM((L,), jnp.float32),
    ],
    compiler_params=_cparams,
)(_polar_body)


def kernel(mu0, mu, edge_idx):
    t = jnp.concatenate(
        [mu0, mu, jnp.zeros((N, 2), jnp.float32)], axis=1)
    pad = EP - E
    i2d = jnp.pad(edge_idx[0], (0, pad)).reshape(EP // 128, 128)
    j2d = jnp.pad(edge_idx[1], (0, pad)).reshape(EP // 128, 128)
    z2d = jnp.zeros((128, SW), jnp.float32)
    s_part, acc_part = _edge_kernel(t, i2d, j2d, z2d)
    dot_part = _polar_kernel(s_part)
    acc1 = jnp.sum(acc_part[:, :, 0, :])
    acc2 = jnp.sum(acc_part[:, :, 1, :])
    dot = jnp.sum(dot_part)
    return (WEIGHT * (acc1 - 2.0 * dot) / acc2).astype(jnp.float32)
